# baseline (speedup 1.0000x reference)
import functools

import numpy as np
import jax
import jax.numpy as jnp
from jax import lax
from jax.experimental import pallas as pl
from jax.experimental.pallas import tpu as pltpu
from jax.experimental.pallas import tpu_sc as plsc

_I1_VALS = [70273749298880, 38956906369248, 16316086777680, 83297495521792,
            191839786542528, 376992761456332, 221880851359940, 0,
            -16781096230092, -27847728347500, -98222995813580, 0,
            793685538262556]

_ROW = 127
_H = 1024
_D = 13
_PLANE = _H * _D
_HH = _H // 2
_CH = _HH * _D
_NSLOTS = 11


def _sc_body(a1, a2, a3, a4, a5, m, ri, ci, out, buf, obuf, mbuf, ribuf,
             cibuf):
    wid = lax.axis_index("s") * 2 + lax.axis_index("c")
    slot = wid // 2
    h = wid % 2
    r0 = h * _HH

    pltpu.sync_copy(m, mbuf)
    pltpu.sync_copy(ri, ribuf)
    pltpu.sync_copy(ci, cibuf)

    off14 = pl.multiple_of(_ROW * _PLANE + h * _CH, 8)
    for k, ref in enumerate((a1, a2, a3, a4)):
        @pl.when(slot == k)
        def _(ref=ref):
            pltpu.sync_copy(ref.at[pl.ds(off14, _CH)], buf)

    @pl.when((slot >= 4) & (slot < _NSLOTS))
    def _():
        off5 = pl.multiple_of(
            ((slot - 4) * 128 + _ROW) * _PLANE + h * _CH, 8)
        pltpu.sync_copy(a5.at[pl.ds(off5, _CH)], buf)

    def body(g, carry):
        s = pl.ds(pl.multiple_of(g * 16, 16), 16)
        v = buf[s] * mbuf[s]
        plsc.store_scatter(obuf, [ribuf[s], cibuf[s]], v)
        return carry

    lax.fori_loop(0, _CH // 16, body, 0)

    for k in range(_NSLOTS):
        @pl.when(slot == k)
        def _(k=k):
            pltpu.sync_copy(obuf, out.at[k, 0, pl.ds(r0, _HH), :])


_sc_call = functools.partial(
    pl.kernel,
    _sc_body,
    out_type=jax.ShapeDtypeStruct((_NSLOTS, 1, _H, _D), jnp.float32),
    mesh=plsc.VectorSubcoreMesh(core_axis_name="c", subcore_axis_name="s",
                                num_cores=2, num_subcores=16),
    compiler_params=pltpu.CompilerParams(needs_layout_passes=False),
    scratch_types=[
        pltpu.VMEM((_CH,), jnp.float32),
        pltpu.VMEM((_HH, _D), jnp.float32),
        pltpu.VMEM((_CH,), jnp.float32),
        pltpu.VMEM((_CH,), jnp.int32),
        pltpu.VMEM((_CH,), jnp.int32),
    ],
)()


def kernel(x1, x2, x3, x4, x5, size):
    del size
    m = jnp.asarray(np.tile(np.asarray(_I1_VALS, dtype=np.float32),
                            _CH // _D))
    f = np.arange(_CH, dtype=np.int32)
    ri = jnp.asarray(f // _D)
    ci = jnp.asarray(f % _D)
    return _sc_call(
        x1.reshape(-1), x2.reshape(-1), x3.reshape(-1), x4.reshape(-1),
        x5.reshape(-1), m, ri, ci)


# speedup vs baseline: 2.7635x; 2.7635x over previous
import functools

import numpy as np
import jax
import jax.numpy as jnp
from jax import lax
from jax.experimental import pallas as pl
from jax.experimental.pallas import tpu as pltpu
from jax.experimental.pallas import tpu_sc as plsc

_I1_VALS = [70273749298880, 38956906369248, 16316086777680, 83297495521792,
            191839786542528, 376992761456332, 221880851359940, 0,
            -16781096230092, -27847728347500, -98222995813580, 0,
            793685538262556]

_ROW = 127
_H = 1024
_D = 13
_PLANE = _H * _D
_HH = _H // 2
_CH = _HH * _D
_NSLOTS = 11


def _sc_body(a1, a2, a3, a4, a5, m, ri, ci, out, buf, obuf, mbuf, ribuf,
             cibuf):
    wid = lax.axis_index("s") * 2 + lax.axis_index("c")
    slot = wid // 2
    h = wid % 2
    r0 = h * _HH
    c0 = h * _CH

    pltpu.sync_copy(m, mbuf)
    pltpu.sync_copy(ri, ribuf)
    pltpu.sync_copy(ci, cibuf)

    for k, ref in enumerate((a1, a2, a3, a4)):
        @pl.when(slot == k)
        def _(ref=ref):
            pltpu.sync_copy(ref.at[_ROW, pl.ds(c0, _CH)], buf)

    for k in range(7):
        @pl.when(slot == 4 + k)
        def _(k=k):
            pltpu.sync_copy(a5.at[k * 128 + _ROW, pl.ds(c0, _CH)], buf)

    def body(g, carry):
        s = pl.ds(pl.multiple_of(g * 16, 16), 16)
        v = buf[s] * mbuf[s]
        plsc.store_scatter(obuf, [ribuf[s], cibuf[s]], v)
        return carry

    lax.fori_loop(0, _CH // 16, body, 0)

    for k in range(_NSLOTS):
        @pl.when(slot == k)
        def _(k=k):
            pltpu.sync_copy(obuf, out.at[k, 0, pl.ds(r0, _HH), :])


_sc_call = functools.partial(
    pl.kernel,
    _sc_body,
    out_type=jax.ShapeDtypeStruct((_NSLOTS, 1, _H, _D), jnp.float32),
    mesh=plsc.VectorSubcoreMesh(core_axis_name="c", subcore_axis_name="s",
                                num_cores=2, num_subcores=16),
    compiler_params=pltpu.CompilerParams(needs_layout_passes=False,
                                         use_tc_tiling_on_sc=True),
    scratch_types=[
        pltpu.VMEM((_CH,), jnp.float32),
        pltpu.VMEM((_HH, _D), jnp.float32),
        pltpu.VMEM((_CH,), jnp.float32),
        pltpu.VMEM((_CH,), jnp.int32),
        pltpu.VMEM((_CH,), jnp.int32),
    ],
)()


def kernel(x1, x2, x3, x4, x5, size):
    del size
    m = jnp.asarray(np.tile(np.asarray(_I1_VALS, dtype=np.float32),
                            _CH // _D))
    f = np.arange(_CH, dtype=np.int32)
    ri = jnp.asarray(f // _D)
    ci = jnp.asarray(f % _D)
    return _sc_call(
        x1.reshape(128, _PLANE), x2.reshape(128, _PLANE),
        x3.reshape(128, _PLANE), x4.reshape(128, _PLANE),
        x5.reshape(7 * 128, _PLANE), m, ri, ci)
